# baseline (speedup 1.0000x reference)
import functools

import jax
import jax.numpy as jnp
from jax import lax
from jax.experimental import pallas as pl
from jax.experimental.pallas import tpu as pltpu
from jax.experimental.pallas import tpu_sc as plsc

N_NODES = 10000
IN_DIM = 384
HID = 128
NUM_SKILLS = 100
NUM_GRAPHS = 64
N_EDGES = 160000

NC, NS = 2, 16
NW = NC * NS
CHUNK = 128
EDGES_PAD = 163840
CH_W = EDGES_PAD // (NW * CHUNK)
NBUF = 2
LOOKAHEAD = 1
CH_RUN = 42
CH_IDX = 44
ACC_ROWS = 10240
DUMMY_ROW = N_NODES
ZROWS = ACC_ROWS // NS
OROWS = 624
OREM = N_NODES - NS * OROWS

ROW_BLK = 2000
N_BLKS = N_NODES // ROW_BLK

_sc_mesh = plsc.VectorSubcoreMesh(
    core_axis_name="c", subcore_axis_name="s", num_cores=NC, num_subcores=NS)


@functools.partial(
    pl.kernel,
    out_type=jax.ShapeDtypeStruct((NC, N_NODES, HID), jnp.float32),
    mesh=_sc_mesh,
    scratch_types=[
        pltpu.VMEM_SHARED((ACC_ROWS, HID), jnp.float32),
        pltpu.VMEM((CH_IDX, CHUNK), jnp.int32),
        pltpu.VMEM((CH_IDX, CHUNK), jnp.int32),
        pltpu.VMEM((NBUF, CHUNK, HID), jnp.float32),
    ] + [pltpu.SemaphoreType.DMA] * (2 * NBUF),
)
def _seg_sum(rows_hbm, src_hbm, dst_hbm, zeros_hbm, out_hbm,
             acc, src_v, dst_v, rows_v, g0, g1, s0, s1):
    gsem = (g0, g1)
    ssem = (s0, s1)
    c = lax.axis_index("c")
    s = lax.axis_index("s")
    wid = s * NC + c
    pltpu.sync_copy(zeros_hbm, acc.at[pl.ds(s * ZROWS, ZROWS)])
    pltpu.sync_copy(src_hbm.at[wid], src_v)
    pltpu.sync_copy(dst_hbm.at[wid], dst_v)
    plsc.subcore_barrier()

    def step(t, b, first):
        bs = (b + LOOKAHEAD) % NBUF
        if not first:
            pltpu.make_async_copy(
                rows_v.at[bs],
                acc.at[dst_v.at[jnp.maximum(t - LOOKAHEAD, 0)]],
                ssem[bs]).wait()
        pltpu.async_copy(rows_hbm.at[src_v.at[t + LOOKAHEAD]], rows_v.at[bs],
                         gsem[bs])
        pltpu.make_async_copy(rows_hbm.at[src_v.at[t]], rows_v.at[b],
                              gsem[b]).wait()
        pltpu.async_copy(rows_v.at[b], acc.at[dst_v.at[t]], ssem[b], add=True)

    for b in range(LOOKAHEAD):
        pltpu.async_copy(rows_hbm.at[src_v.at[b]], rows_v.at[b], gsem[b])
    for b in range(NBUF):
        step(b, b, first=b < LOOKAHEAD)

    def ring_round(jj, carry):
        for b in range(NBUF):
            step(jj * NBUF + b, b, first=False)
        return carry

    lax.fori_loop(1, CH_RUN // NBUF, ring_round, 0)
    for b in range(LOOKAHEAD):
        bb = (CH_RUN - LOOKAHEAD + b) % NBUF
        pltpu.make_async_copy(rows_v.at[bb], acc.at[dst_v.at[0]],
                              ssem[bb]).wait()
        bg = (CH_RUN + b) % NBUF
        pltpu.make_async_copy(rows_hbm.at[src_v.at[CH_RUN + b]],
                              rows_v.at[bg], gsem[bg]).wait()
    plsc.subcore_barrier()
    pltpu.sync_copy(acc.at[pl.ds(s * OROWS, OROWS)],
                    out_hbm.at[c, pl.ds(s * OROWS, OROWS)])

    @pl.when(s == NS - 1)
    def _():
        pltpu.sync_copy(acc.at[pl.ds(NS * OROWS, OREM)],
                        out_hbm.at[c, pl.ds(NS * OROWS, OREM)])


def _stage1_body(x_ref, s00, s01, s10, s11, s20, s21, w1_ref, b1_ref, w2_ref,
                 b2_ref, o_ref):
    agg = jnp.concatenate(
        [s00[...] + s01[...], s10[...] + s11[...], s20[...] + s21[...]],
        axis=1)
    hin = x_ref[...] + agg
    h = jnp.dot(hin, w1_ref[...], preferred_element_type=jnp.float32)
    h = jnp.maximum(h + b1_ref[...], 0.0)
    t = jnp.dot(h, w2_ref[...], preferred_element_type=jnp.float32)
    o_ref[...] = jnp.maximum(t + b2_ref[...], 0.0)


def _stage2_body(t_ref, a0_ref, a1_ref, w3_ref, b3_ref, w4_ref, b4_ref,
                 batch_ref, wc_ref, bc_ref, o_ref, acc_ref):
    i = pl.program_id(0)

    @pl.when(i == 0)
    def _():
        acc_ref[...] = jnp.zeros_like(acc_ref)

    hin = t_ref[...] + a0_ref[...] + a1_ref[...]
    h = jnp.dot(hin, w3_ref[...], preferred_element_type=jnp.float32)
    h = jnp.maximum(h + b3_ref[...], 0.0)
    t = jnp.dot(h, w4_ref[...], preferred_element_type=jnp.float32)
    t = jnp.maximum(t + b4_ref[...], 0.0)
    gids = lax.broadcasted_iota(jnp.int32, (NUM_GRAPHS, ROW_BLK), 0)
    oh = (jnp.broadcast_to(batch_ref[0], (NUM_GRAPHS, ROW_BLK)) == gids)
    pooled = lax.dot_general(oh.astype(jnp.float32), t,
                             (((1,), (0,)), ((), ())),
                             preferred_element_type=jnp.float32,
                             precision=lax.Precision.HIGHEST)
    acc_ref[...] += pooled

    @pl.when(i == N_BLKS - 1)
    def _():
        logits = jnp.dot(acc_ref[...], wc_ref[...],
                         preferred_element_type=jnp.float32) + bc_ref[...]
        o_ref[...] = jax.nn.sigmoid(logits)


def _stage1(x, slabs, w1, b1, w2, b2):
    row = pl.BlockSpec((ROW_BLK, HID), lambda i: (i, 0))
    return pl.pallas_call(
        _stage1_body,
        grid=(N_BLKS,),
        in_specs=[
            pl.BlockSpec((ROW_BLK, IN_DIM), lambda i: (i, 0)),
            row, row, row, row, row, row,
            pl.BlockSpec((IN_DIM, HID), lambda i: (0, 0)),
            pl.BlockSpec((HID,), lambda i: (0,)),
            pl.BlockSpec((HID, HID), lambda i: (0, 0)),
            pl.BlockSpec((HID,), lambda i: (0,)),
        ],
        out_specs=pl.BlockSpec((ROW_BLK, HID), lambda i: (i, 0)),
        out_shape=jax.ShapeDtypeStruct((N_NODES, HID), jnp.float32),
    )(x, slabs[0][0], slabs[0][1], slabs[1][0], slabs[1][1], slabs[2][0],
      slabs[2][1], w1, b1, w2, b2)


def _stage2(t1, a0, a1, w3, b3, w4, b4, batch3, wc, bc):
    row = pl.BlockSpec((ROW_BLK, HID), lambda i: (i, 0))
    full_v = pl.BlockSpec((HID,), lambda i: (0,))
    full_m = pl.BlockSpec((HID, HID), lambda i: (0, 0))
    return pl.pallas_call(
        _stage2_body,
        grid=(N_BLKS,),
        in_specs=[
            row, row, row, full_m, full_v, full_m, full_v,
            pl.BlockSpec((1, 1, ROW_BLK), lambda i: (i, 0, 0)),
            pl.BlockSpec((HID, NUM_SKILLS), lambda i: (0, 0)),
            pl.BlockSpec((NUM_SKILLS,), lambda i: (0,)),
        ],
        out_specs=pl.BlockSpec((NUM_GRAPHS, NUM_SKILLS), lambda i: (0, 0)),
        out_shape=jax.ShapeDtypeStruct((NUM_GRAPHS, NUM_SKILLS), jnp.float32),
        scratch_shapes=[pltpu.VMEM((NUM_GRAPHS, HID), jnp.float32)],
    )(t1, a0, a1, w3, b3, w4, b4, batch3, wc, bc)


def kernel(x, edge_index, batch, W1, b1, W2, b2, W3, b3, W4, b4, Wc, bc):
    x = x.astype(jnp.float32)
    src = edge_index[0].astype(jnp.int32)
    dst = edge_index[1].astype(jnp.int32)
    pad = EDGES_PAD - src.shape[0]
    src_p = jnp.concatenate([src, jnp.zeros((pad,), jnp.int32)])
    dst_p = jnp.concatenate([dst, jnp.full((pad,), DUMMY_ROW, jnp.int32)])
    pad_ch = CH_IDX - CH_W

    def chunked(idx_flat, fill):
        w = idx_flat.reshape(NW, CH_W, CHUNK)
        return jnp.concatenate(
            [w, jnp.full((NW, pad_ch, CHUNK), fill, jnp.int32)], axis=1)

    dst_w = chunked(dst_p, DUMMY_ROW)
    src3 = 3 * src_p
    x3 = x.reshape(3 * N_NODES, HID)
    zeros = jnp.zeros((ZROWS, HID), jnp.float32)
    batch3 = batch.astype(jnp.int32).reshape(N_BLKS, 1, ROW_BLK)

    slabs = [_seg_sum(x3, chunked(src3 + k, 0), dst_w, zeros)
             for k in range(3)]
    t1 = _stage1(x, slabs, W1, b1, W2, b2)
    a2 = _seg_sum(t1, chunked(src_p, 0), dst_w, zeros)
    return _stage2(t1, a2[0], a2[1], W3, b3, W4, b4, batch3, Wc, bc)


# speedup vs baseline: 1.2929x; 1.2929x over previous
import functools

import jax
import jax.numpy as jnp
from jax import lax
from jax.experimental import pallas as pl
from jax.experimental.pallas import tpu as pltpu
from jax.experimental.pallas import tpu_sc as plsc

N_NODES = 10000
IN_DIM = 384
HID = 128
NUM_SKILLS = 100
NUM_GRAPHS = 64
N_EDGES = 160000

NC, NS = 2, 16
NW = NC * NS
CHUNK = 128
EDGES_PAD = 163840
CH_W = EDGES_PAD // (NW * CHUNK)
NBUF = 2
LOOKAHEAD = 1
CH_RUN = 42
CH_IDX = 44
ACC_ROWS = 10240
DUMMY_ROW = N_NODES
ZROWS = ACC_ROWS // NS
OROWS = 624
OREM = N_NODES - NS * OROWS

ROW_BLK = 2000
N_BLKS = N_NODES // ROW_BLK

_sc_mesh = plsc.VectorSubcoreMesh(
    core_axis_name="c", subcore_axis_name="s", num_cores=NC, num_subcores=NS)


@functools.partial(
    pl.kernel,
    out_type=jax.ShapeDtypeStruct((NC, N_NODES, HID), jnp.float32),
    mesh=_sc_mesh,
    scratch_types=[
        pltpu.VMEM_SHARED((ACC_ROWS, HID), jnp.float32),
        pltpu.VMEM((CH_IDX, CHUNK), jnp.int32),
        pltpu.VMEM((CH_IDX, CHUNK), jnp.int32),
        pltpu.VMEM((NBUF, CHUNK, HID), jnp.float32),
    ] + [pltpu.SemaphoreType.DMA] * (2 * NBUF),
)
def _seg_sum(rows_hbm, src_hbm, dst_hbm, zeros_hbm, out_hbm,
             acc, src_v, dst_v, rows_v, g0, g1, s0, s1):
    gsem = (g0, g1)
    ssem = (s0, s1)
    c = lax.axis_index("c")
    s = lax.axis_index("s")
    wid = s * NC + c
    pltpu.sync_copy(zeros_hbm, acc.at[pl.ds(s * ZROWS, ZROWS)])
    pltpu.sync_copy(src_hbm.at[wid], src_v)
    pltpu.sync_copy(dst_hbm.at[wid], dst_v)
    plsc.subcore_barrier()

    def fire_gather(t, b):
        pltpu.async_copy(rows_hbm.at[src_v.at[t]], rows_v.at[b], gsem[b])

    def wait_gather(t, b):
        pltpu.make_async_copy(rows_hbm.at[src_v.at[t]], rows_v.at[b],
                              gsem[b]).wait()

    def fire_scatter(t, b):
        pltpu.async_copy(rows_v.at[b], acc.at[dst_v.at[t]], ssem[b],
                         add=True)

    def wait_scatter(t, b):
        pltpu.make_async_copy(rows_v.at[b], acc.at[dst_v.at[t]],
                              ssem[b]).wait()

    fire_gather(0, 0)
    fire_gather(1, 1)
    wait_gather(0, 0)
    fire_scatter(0, 0)
    wait_gather(1, 1)
    fire_scatter(1, 1)

    def pair(j, carry):
        t0 = 2 * j
        wait_scatter(t0 - 2, 0)
        fire_gather(t0, 0)
        wait_scatter(t0 - 1, 1)
        fire_gather(t0 + 1, 1)
        wait_gather(t0, 0)
        fire_scatter(t0, 0)
        wait_gather(t0 + 1, 1)
        fire_scatter(t0 + 1, 1)
        return carry

    lax.fori_loop(1, CH_RUN // 2, pair, 0)
    wait_scatter(CH_RUN - 2, 0)
    wait_scatter(CH_RUN - 1, 1)
    plsc.subcore_barrier()
    pltpu.sync_copy(acc.at[pl.ds(s * OROWS, OROWS)],
                    out_hbm.at[c, pl.ds(s * OROWS, OROWS)])

    @pl.when(s == NS - 1)
    def _():
        pltpu.sync_copy(acc.at[pl.ds(NS * OROWS, OREM)],
                        out_hbm.at[c, pl.ds(NS * OROWS, OREM)])


def _stage1_body(x_ref, s00, s01, s10, s11, s20, s21, w1_ref, b1_ref, w2_ref,
                 b2_ref, o_ref):
    agg = jnp.concatenate(
        [s00[...] + s01[...], s10[...] + s11[...], s20[...] + s21[...]],
        axis=1)
    hin = x_ref[...] + agg
    h = jnp.dot(hin, w1_ref[...], preferred_element_type=jnp.float32)
    h = jnp.maximum(h + b1_ref[...], 0.0)
    t = jnp.dot(h, w2_ref[...], preferred_element_type=jnp.float32)
    o_ref[...] = jnp.maximum(t + b2_ref[...], 0.0)


def _stage2_body(t_ref, a0_ref, a1_ref, w3_ref, b3_ref, w4_ref, b4_ref,
                 batch_ref, wc_ref, bc_ref, o_ref, acc_ref):
    i = pl.program_id(0)

    @pl.when(i == 0)
    def _():
        acc_ref[...] = jnp.zeros_like(acc_ref)

    hin = t_ref[...] + a0_ref[...] + a1_ref[...]
    h = jnp.dot(hin, w3_ref[...], preferred_element_type=jnp.float32)
    h = jnp.maximum(h + b3_ref[...], 0.0)
    t = jnp.dot(h, w4_ref[...], preferred_element_type=jnp.float32)
    t = jnp.maximum(t + b4_ref[...], 0.0)
    gids = lax.broadcasted_iota(jnp.int32, (NUM_GRAPHS, ROW_BLK), 0)
    oh = (jnp.broadcast_to(batch_ref[0], (NUM_GRAPHS, ROW_BLK)) == gids)
    pooled = lax.dot_general(oh.astype(jnp.float32), t,
                             (((1,), (0,)), ((), ())),
                             preferred_element_type=jnp.float32,
                             precision=lax.Precision.HIGHEST)
    acc_ref[...] += pooled

    @pl.when(i == N_BLKS - 1)
    def _():
        logits = jnp.dot(acc_ref[...], wc_ref[...],
                         preferred_element_type=jnp.float32) + bc_ref[...]
        o_ref[...] = jax.nn.sigmoid(logits)


def _stage1(x, slabs, w1, b1, w2, b2):
    row = pl.BlockSpec((ROW_BLK, HID), lambda i: (i, 0))
    return pl.pallas_call(
        _stage1_body,
        grid=(N_BLKS,),
        in_specs=[
            pl.BlockSpec((ROW_BLK, IN_DIM), lambda i: (i, 0)),
            row, row, row, row, row, row,
            pl.BlockSpec((IN_DIM, HID), lambda i: (0, 0)),
            pl.BlockSpec((HID,), lambda i: (0,)),
            pl.BlockSpec((HID, HID), lambda i: (0, 0)),
            pl.BlockSpec((HID,), lambda i: (0,)),
        ],
        out_specs=pl.BlockSpec((ROW_BLK, HID), lambda i: (i, 0)),
        out_shape=jax.ShapeDtypeStruct((N_NODES, HID), jnp.float32),
    )(x, slabs[0][0], slabs[0][1], slabs[1][0], slabs[1][1], slabs[2][0],
      slabs[2][1], w1, b1, w2, b2)


def _stage2(t1, a0, a1, w3, b3, w4, b4, batch3, wc, bc):
    row = pl.BlockSpec((ROW_BLK, HID), lambda i: (i, 0))
    full_v = pl.BlockSpec((HID,), lambda i: (0,))
    full_m = pl.BlockSpec((HID, HID), lambda i: (0, 0))
    return pl.pallas_call(
        _stage2_body,
        grid=(N_BLKS,),
        in_specs=[
            row, row, row, full_m, full_v, full_m, full_v,
            pl.BlockSpec((1, 1, ROW_BLK), lambda i: (i, 0, 0)),
            pl.BlockSpec((HID, NUM_SKILLS), lambda i: (0, 0)),
            pl.BlockSpec((NUM_SKILLS,), lambda i: (0,)),
        ],
        out_specs=pl.BlockSpec((NUM_GRAPHS, NUM_SKILLS), lambda i: (0, 0)),
        out_shape=jax.ShapeDtypeStruct((NUM_GRAPHS, NUM_SKILLS), jnp.float32),
        scratch_shapes=[pltpu.VMEM((NUM_GRAPHS, HID), jnp.float32)],
    )(t1, a0, a1, w3, b3, w4, b4, batch3, wc, bc)


def kernel(x, edge_index, batch, W1, b1, W2, b2, W3, b3, W4, b4, Wc, bc):
    x = x.astype(jnp.float32)
    src = edge_index[0].astype(jnp.int32)
    dst = edge_index[1].astype(jnp.int32)
    pad = EDGES_PAD - src.shape[0]
    src_p = jnp.concatenate([src, jnp.zeros((pad,), jnp.int32)])
    dst_p = jnp.concatenate([dst, jnp.full((pad,), DUMMY_ROW, jnp.int32)])
    pad_ch = CH_IDX - CH_W

    def chunked(idx_flat, fill):
        w = idx_flat.reshape(NW, CH_W, CHUNK)
        return jnp.concatenate(
            [w, jnp.full((NW, pad_ch, CHUNK), fill, jnp.int32)], axis=1)

    dst_w = chunked(dst_p, DUMMY_ROW)
    src3 = 3 * src_p
    x3 = x.reshape(3 * N_NODES, HID)
    zeros = jnp.zeros((ZROWS, HID), jnp.float32)
    batch3 = batch.astype(jnp.int32).reshape(N_BLKS, 1, ROW_BLK)

    slabs = [_seg_sum(x3, chunked(src3 + k, 0), dst_w, zeros)
             for k in range(3)]
    t1 = _stage1(x, slabs, W1, b1, W2, b2)
    a2 = _seg_sum(t1, chunked(src_p, 0), dst_w, zeros)
    return _stage2(t1, a2[0], a2[1], W3, b3, W4, b4, batch3, Wc, bc)


# speedup vs baseline: 2.1536x; 1.6657x over previous
import functools

import jax
import jax.numpy as jnp
from jax import lax
from jax.experimental import pallas as pl
from jax.experimental.pallas import tpu as pltpu
from jax.experimental.pallas import tpu_sc as plsc

N_NODES = 10000
IN_DIM = 384
HID = 128
NUM_SKILLS = 100
NUM_GRAPHS = 64
N_EDGES = 160000

NC, NS = 2, 16
NW = NC * NS
CHUNK = 128
EDGES_PAD = 163840
CH_W = EDGES_PAD // (NW * CHUNK)
ACC_ROWS = 10240
DUMMY_ROW = N_NODES
ZROWS = ACC_ROWS // NS
OROWS = 624
OREM = N_NODES - NS * OROWS

ROW_BLK = 2000
N_BLKS = N_NODES // ROW_BLK

_sc_mesh = plsc.VectorSubcoreMesh(
    core_axis_name="c", subcore_axis_name="s", num_cores=NC, num_subcores=NS)


def _make_seg_sum(nslabs):
    @functools.partial(
        pl.kernel,
        out_type=jax.ShapeDtypeStruct((NC, N_NODES, nslabs * HID),
                                      jnp.float32),
        mesh=_sc_mesh,
        scratch_types=[
            pltpu.VMEM_SHARED((ACC_ROWS, HID), jnp.float32),
            pltpu.VMEM((CH_W, CHUNK), jnp.int32),
            pltpu.VMEM((CH_W, CHUNK), jnp.int32),
            pltpu.VMEM((2, CHUNK, HID), jnp.float32),
            pltpu.SemaphoreType.DMA,
            pltpu.SemaphoreType.DMA,
        ],
    )
    def _seg(rows_hbm, src_hbm, dst_hbm, zeros_hbm, out_hbm,
             acc, src_v, dst_v, rows_v, sem0, sem1):
        c = lax.axis_index("c")
        s = lax.axis_index("s")
        wid = s * NC + c
        pltpu.sync_copy(dst_hbm.at[wid], dst_v)
        for k in range(nslabs):
            pltpu.sync_copy(src_hbm.at[wid, k], src_v)
            pltpu.sync_copy(zeros_hbm, acc.at[pl.ds(s * ZROWS, ZROWS)])
            plsc.subcore_barrier()

            def chunk_pair(j, carry):
                i0 = 2 * j
                g0 = pltpu.async_copy(rows_hbm.at[src_v.at[i0]],
                                      rows_v.at[0], sem0)
                g1 = pltpu.async_copy(rows_hbm.at[src_v.at[i0 + 1]],
                                      rows_v.at[1], sem1)
                g0.wait()
                pltpu.sync_copy(rows_v.at[0], acc.at[dst_v.at[i0]], add=True)
                g1.wait()
                pltpu.sync_copy(rows_v.at[1], acc.at[dst_v.at[i0 + 1]],
                                add=True)
                return carry

            lax.fori_loop(0, CH_W // 2, chunk_pair, 0)
            plsc.subcore_barrier()
            pltpu.sync_copy(
                acc.at[pl.ds(s * OROWS, OROWS)],
                out_hbm.at[c, pl.ds(s * OROWS, OROWS), pl.ds(k * HID, HID)])

            @pl.when(s == NS - 1)
            def _():
                pltpu.sync_copy(
                    acc.at[pl.ds(NS * OROWS, OREM)],
                    out_hbm.at[c, pl.ds(NS * OROWS, OREM),
                               pl.ds(k * HID, HID)])

            if k + 1 < nslabs:
                plsc.subcore_barrier()

    return _seg


_seg_sum_l1 = _make_seg_sum(IN_DIM // HID)
_seg_sum_l2 = _make_seg_sum(1)


def _stage1_body(x_ref, a0_ref, a1_ref, w1_ref, b1_ref, w2_ref,
                 b2_ref, o_ref):
    hin = x_ref[...] + a0_ref[...] + a1_ref[...]
    h = jnp.dot(hin, w1_ref[...], preferred_element_type=jnp.float32)
    h = jnp.maximum(h + b1_ref[...], 0.0)
    t = jnp.dot(h, w2_ref[...], preferred_element_type=jnp.float32)
    o_ref[...] = jnp.maximum(t + b2_ref[...], 0.0)


def _stage2_body(t_ref, a0_ref, a1_ref, w3_ref, b3_ref, w4_ref, b4_ref,
                 batch_ref, wc_ref, bc_ref, o_ref, acc_ref):
    i = pl.program_id(0)

    @pl.when(i == 0)
    def _():
        acc_ref[...] = jnp.zeros_like(acc_ref)

    hin = t_ref[...] + a0_ref[...] + a1_ref[...]
    h = jnp.dot(hin, w3_ref[...], preferred_element_type=jnp.float32)
    h = jnp.maximum(h + b3_ref[...], 0.0)
    t = jnp.dot(h, w4_ref[...], preferred_element_type=jnp.float32)
    t = jnp.maximum(t + b4_ref[...], 0.0)
    gids = lax.broadcasted_iota(jnp.int32, (NUM_GRAPHS, ROW_BLK), 0)
    oh = (jnp.broadcast_to(batch_ref[0], (NUM_GRAPHS, ROW_BLK)) == gids)
    pooled = lax.dot_general(oh.astype(jnp.float32), t,
                             (((1,), (0,)), ((), ())),
                             preferred_element_type=jnp.float32,
                             precision=lax.Precision.HIGHEST)
    acc_ref[...] += pooled

    @pl.when(i == N_BLKS - 1)
    def _():
        logits = jnp.dot(acc_ref[...], wc_ref[...],
                         preferred_element_type=jnp.float32) + bc_ref[...]
        o_ref[...] = jax.nn.sigmoid(logits)


def _stage1(x, a0, a1, w1, b1, w2, b2):
    wide = pl.BlockSpec((ROW_BLK, IN_DIM), lambda i: (i, 0))
    return pl.pallas_call(
        _stage1_body,
        grid=(N_BLKS,),
        in_specs=[
            wide, wide, wide,
            pl.BlockSpec((IN_DIM, HID), lambda i: (0, 0)),
            pl.BlockSpec((HID,), lambda i: (0,)),
            pl.BlockSpec((HID, HID), lambda i: (0, 0)),
            pl.BlockSpec((HID,), lambda i: (0,)),
        ],
        out_specs=pl.BlockSpec((ROW_BLK, HID), lambda i: (i, 0)),
        out_shape=jax.ShapeDtypeStruct((N_NODES, HID), jnp.float32),
    )(x, a0, a1, w1, b1, w2, b2)


def _stage2(t1, a0, a1, w3, b3, w4, b4, batch3, wc, bc):
    row = pl.BlockSpec((ROW_BLK, HID), lambda i: (i, 0))
    full_v = pl.BlockSpec((HID,), lambda i: (0,))
    full_m = pl.BlockSpec((HID, HID), lambda i: (0, 0))
    return pl.pallas_call(
        _stage2_body,
        grid=(N_BLKS,),
        in_specs=[
            row, row, row, full_m, full_v, full_m, full_v,
            pl.BlockSpec((1, 1, ROW_BLK), lambda i: (i, 0, 0)),
            pl.BlockSpec((HID, NUM_SKILLS), lambda i: (0, 0)),
            pl.BlockSpec((NUM_SKILLS,), lambda i: (0,)),
        ],
        out_specs=pl.BlockSpec((NUM_GRAPHS, NUM_SKILLS), lambda i: (0, 0)),
        out_shape=jax.ShapeDtypeStruct((NUM_GRAPHS, NUM_SKILLS), jnp.float32),
        scratch_shapes=[pltpu.VMEM((NUM_GRAPHS, HID), jnp.float32)],
    )(t1, a0, a1, w3, b3, w4, b4, batch3, wc, bc)


def kernel(x, edge_index, batch, W1, b1, W2, b2, W3, b3, W4, b4, Wc, bc):
    x = x.astype(jnp.float32)
    src = edge_index[0].astype(jnp.int32)
    dst = edge_index[1].astype(jnp.int32)
    pad = EDGES_PAD - src.shape[0]
    src_p = jnp.concatenate([src, jnp.zeros((pad,), jnp.int32)])
    dst_p = jnp.concatenate([dst, jnp.full((pad,), DUMMY_ROW, jnp.int32)])
    dst_w = dst_p.reshape(NW, CH_W, CHUNK)
    src_l1 = jnp.stack([3 * src_p + k for k in range(3)], axis=1).reshape(
        NW, CH_W * CHUNK, 3).transpose(0, 2, 1).reshape(NW, 3, CH_W, CHUNK)
    src_l2 = src_p.reshape(NW, 1, CH_W, CHUNK)
    x3 = x.reshape(3 * N_NODES, HID)
    zeros = jnp.zeros((ZROWS, HID), jnp.float32)
    batch3 = batch.astype(jnp.int32).reshape(N_BLKS, 1, ROW_BLK)

    a1 = _seg_sum_l1(x3, src_l1, dst_w, zeros)
    t1 = _stage1(x, a1[0], a1[1], W1, b1, W2, b2)
    a2 = _seg_sum_l2(t1, src_l2, dst_w, zeros)
    return _stage2(t1, a2[0], a2[1], W3, b3, W4, b4, batch3, Wc, bc)


# speedup vs baseline: 2.3013x; 1.0686x over previous
import functools

import jax
import jax.numpy as jnp
from jax import lax
from jax.experimental import pallas as pl
from jax.experimental.pallas import tpu as pltpu
from jax.experimental.pallas import tpu_sc as plsc

N_NODES = 10000
IN_DIM = 384
HID = 128
NUM_SKILLS = 100
NUM_GRAPHS = 64
N_EDGES = 160000

NC, NS = 2, 16
NW = NC * NS
CHUNK = 128
EDGES_PAD = 163840
CH_W = EDGES_PAD // (NW * CHUNK)
ACC_ROWS = 10240
DUMMY_ROW = N_NODES
ZROWS = ACC_ROWS // NS
OROWS = 624
OREM = N_NODES - NS * OROWS

ROW_BLK = 2000
N_BLKS = N_NODES // ROW_BLK

_sc_mesh = plsc.VectorSubcoreMesh(
    core_axis_name="c", subcore_axis_name="s", num_cores=NC, num_subcores=NS)


def _make_seg_sum(nslabs):
    @functools.partial(
        pl.kernel,
        out_type=jax.ShapeDtypeStruct((NC, N_NODES, nslabs * HID),
                                      jnp.float32),
        mesh=_sc_mesh,
        scratch_types=[
            pltpu.VMEM_SHARED((ACC_ROWS, HID), jnp.float32),
            pltpu.VMEM((CH_W, CHUNK), jnp.int32),
            pltpu.VMEM((CH_W, CHUNK), jnp.int32),
            pltpu.VMEM((2, CHUNK, HID), jnp.float32),
            pltpu.SemaphoreType.DMA,
            pltpu.SemaphoreType.DMA,
            pltpu.SemaphoreType.DMA,
            pltpu.SemaphoreType.DMA,
        ],
    )
    def _seg(rows_hbm, src_hbm, dst_hbm, zeros_hbm, out_hbm,
             acc, src_v, dst_v, rows_v, sem0, sem1, sem2, sem3):
        c = lax.axis_index("c")
        s = lax.axis_index("s")
        wid = s * NC + c
        pltpu.sync_copy(dst_hbm.at[wid], dst_v)
        for k in range(nslabs):
            pltpu.sync_copy(src_hbm.at[wid, k], src_v)
            pltpu.sync_copy(zeros_hbm, acc.at[pl.ds(s * ZROWS, ZROWS)])
            plsc.subcore_barrier()

            def chunk_pair(j, carry):
                i0 = 2 * j
                g0 = pltpu.async_copy(rows_hbm.at[src_v.at[i0]],
                                      rows_v.at[0], sem0)
                g1 = pltpu.async_copy(rows_hbm.at[src_v.at[i0 + 1]],
                                      rows_v.at[1], sem1)
                g0.wait()
                s0 = pltpu.async_copy(rows_v.at[0], acc.at[dst_v.at[i0]],
                                      sem2, add=True)
                g1.wait()
                s1 = pltpu.async_copy(rows_v.at[1],
                                      acc.at[dst_v.at[i0 + 1]], sem3,
                                      add=True)
                s0.wait()
                s1.wait()
                return carry

            lax.fori_loop(0, CH_W // 2, chunk_pair, 0)
            plsc.subcore_barrier()
            pltpu.sync_copy(
                acc.at[pl.ds(s * OROWS, OROWS)],
                out_hbm.at[c, pl.ds(s * OROWS, OROWS), pl.ds(k * HID, HID)])

            @pl.when(s == NS - 1)
            def _():
                pltpu.sync_copy(
                    acc.at[pl.ds(NS * OROWS, OREM)],
                    out_hbm.at[c, pl.ds(NS * OROWS, OREM),
                               pl.ds(k * HID, HID)])

            if k + 1 < nslabs:
                plsc.subcore_barrier()

    return _seg


_seg_sum = _make_seg_sum(1)


def _stage1_body(x_ref, s00, s01, s10, s11, s20, s21, w1_ref, b1_ref, w2_ref,
                 b2_ref, o_ref):
    agg = jnp.concatenate(
        [s00[...] + s01[...], s10[...] + s11[...], s20[...] + s21[...]],
        axis=1)
    hin = x_ref[...] + agg
    h = jnp.dot(hin, w1_ref[...], preferred_element_type=jnp.float32)
    h = jnp.maximum(h + b1_ref[...], 0.0)
    t = jnp.dot(h, w2_ref[...], preferred_element_type=jnp.float32)
    o_ref[...] = jnp.maximum(t + b2_ref[...], 0.0)


def _stage2_body(t_ref, a0_ref, a1_ref, w3_ref, b3_ref, w4_ref, b4_ref,
                 batch_ref, wc_ref, bc_ref, o_ref, acc_ref):
    i = pl.program_id(0)

    @pl.when(i == 0)
    def _():
        acc_ref[...] = jnp.zeros_like(acc_ref)

    hin = t_ref[...] + a0_ref[...] + a1_ref[...]
    h = jnp.dot(hin, w3_ref[...], preferred_element_type=jnp.float32)
    h = jnp.maximum(h + b3_ref[...], 0.0)
    t = jnp.dot(h, w4_ref[...], preferred_element_type=jnp.float32)
    t = jnp.maximum(t + b4_ref[...], 0.0)
    gids = lax.broadcasted_iota(jnp.int32, (NUM_GRAPHS, ROW_BLK), 0)
    oh = (jnp.broadcast_to(batch_ref[0], (NUM_GRAPHS, ROW_BLK)) == gids)
    pooled = lax.dot_general(oh.astype(jnp.float32), t,
                             (((1,), (0,)), ((), ())),
                             preferred_element_type=jnp.float32,
                             precision=lax.Precision.HIGHEST)
    acc_ref[...] += pooled

    @pl.when(i == N_BLKS - 1)
    def _():
        logits = jnp.dot(acc_ref[...], wc_ref[...],
                         preferred_element_type=jnp.float32) + bc_ref[...]
        o_ref[...] = jax.nn.sigmoid(logits)


def _stage1(x, slabs, w1, b1, w2, b2):
    row = pl.BlockSpec((ROW_BLK, HID), lambda i: (i, 0))
    return pl.pallas_call(
        _stage1_body,
        grid=(N_BLKS,),
        in_specs=[
            pl.BlockSpec((ROW_BLK, IN_DIM), lambda i: (i, 0)),
            row, row, row, row, row, row,
            pl.BlockSpec((IN_DIM, HID), lambda i: (0, 0)),
            pl.BlockSpec((HID,), lambda i: (0,)),
            pl.BlockSpec((HID, HID), lambda i: (0, 0)),
            pl.BlockSpec((HID,), lambda i: (0,)),
        ],
        out_specs=pl.BlockSpec((ROW_BLK, HID), lambda i: (i, 0)),
        out_shape=jax.ShapeDtypeStruct((N_NODES, HID), jnp.float32),
    )(x, slabs[0][0], slabs[0][1], slabs[1][0], slabs[1][1], slabs[2][0],
      slabs[2][1], w1, b1, w2, b2)


def _stage2(t1, a0, a1, w3, b3, w4, b4, batch3, wc, bc):
    row = pl.BlockSpec((ROW_BLK, HID), lambda i: (i, 0))
    full_v = pl.BlockSpec((HID,), lambda i: (0,))
    full_m = pl.BlockSpec((HID, HID), lambda i: (0, 0))
    return pl.pallas_call(
        _stage2_body,
        grid=(N_BLKS,),
        in_specs=[
            row, row, row, full_m, full_v, full_m, full_v,
            pl.BlockSpec((1, 1, ROW_BLK), lambda i: (i, 0, 0)),
            pl.BlockSpec((HID, NUM_SKILLS), lambda i: (0, 0)),
            pl.BlockSpec((NUM_SKILLS,), lambda i: (0,)),
        ],
        out_specs=pl.BlockSpec((NUM_GRAPHS, NUM_SKILLS), lambda i: (0, 0)),
        out_shape=jax.ShapeDtypeStruct((NUM_GRAPHS, NUM_SKILLS), jnp.float32),
        scratch_shapes=[pltpu.VMEM((NUM_GRAPHS, HID), jnp.float32)],
    )(t1, a0, a1, w3, b3, w4, b4, batch3, wc, bc)


def kernel(x, edge_index, batch, W1, b1, W2, b2, W3, b3, W4, b4, Wc, bc):
    x = x.astype(jnp.float32)
    src = edge_index[0].astype(jnp.int32)
    dst = edge_index[1].astype(jnp.int32)
    pad = EDGES_PAD - src.shape[0]
    src_p = jnp.concatenate([src, jnp.zeros((pad,), jnp.int32)])
    dst_p = jnp.concatenate([dst, jnp.full((pad,), DUMMY_ROW, jnp.int32)])
    dst_w = dst_p.reshape(NW, CH_W, CHUNK)
    src3 = 3 * src_p
    x3 = x.reshape(3 * N_NODES, HID)
    zeros = jnp.zeros((ZROWS, HID), jnp.float32)
    batch3 = batch.astype(jnp.int32).reshape(N_BLKS, 1, ROW_BLK)

    slabs = [_seg_sum(x3, (src3 + k).reshape(NW, 1, CH_W, CHUNK), dst_w,
                      zeros)
             for k in range(3)]
    t1 = _stage1(x, slabs, W1, b1, W2, b2)
    a2 = _seg_sum(t1, src_p.reshape(NW, 1, CH_W, CHUNK), dst_w, zeros)
    return _stage2(t1, a2[0], a2[1], W3, b3, W4, b4, batch3, Wc, bc)


# speedup vs baseline: 2.3024x; 1.0005x over previous
import functools

import jax
import jax.numpy as jnp
from jax import lax
from jax.experimental import pallas as pl
from jax.experimental.pallas import tpu as pltpu
from jax.experimental.pallas import tpu_sc as plsc

N_NODES = 10000
IN_DIM = 384
HID = 128
NUM_SKILLS = 100
NUM_GRAPHS = 64
N_EDGES = 160000

NC, NS = 2, 16
NW = NC * NS
CHUNK = 128
EDGES_PAD = 163840
CH_W = EDGES_PAD // (NW * CHUNK)
ACC_ROWS = 10240
DUMMY_ROW = N_NODES
ZROWS = ACC_ROWS // NS
OROWS = 624
OREM = N_NODES - NS * OROWS

ROW_BLK = 2000
N_BLKS = N_NODES // ROW_BLK

_sc_mesh = plsc.VectorSubcoreMesh(
    core_axis_name="c", subcore_axis_name="s", num_cores=NC, num_subcores=NS)


def _make_seg_sum(nslabs):
    @functools.partial(
        pl.kernel,
        out_type=jax.ShapeDtypeStruct((NC, N_NODES, nslabs * HID),
                                      jnp.float32),
        mesh=_sc_mesh,
        scratch_types=[
            pltpu.VMEM_SHARED((ACC_ROWS, HID), jnp.float32),
            pltpu.VMEM((CH_W, CHUNK), jnp.int32),
            pltpu.VMEM((CH_W, CHUNK), jnp.int32),
            pltpu.VMEM((2, CHUNK, HID), jnp.float32),
        ] + [pltpu.SemaphoreType.DMA] * 4,
    )
    def _seg(rows_hbm, src_hbm, dst_hbm, zeros_hbm, out_hbm,
             acc, src_v, dst_v, rows_v, sem0, sem1, sem2, sem3):
        c = lax.axis_index("c")
        s = lax.axis_index("s")
        wid = s * NC + c
        pltpu.sync_copy(dst_hbm.at[wid], dst_v)
        for k in range(nslabs):
            pltpu.sync_copy(src_hbm.at[wid, k], src_v)
            pltpu.sync_copy(zeros_hbm, acc.at[pl.ds(s * ZROWS, ZROWS)])
            plsc.subcore_barrier()

            def chunk_pair(j, carry):
                i0 = 2 * j
                g0 = pltpu.async_copy(rows_hbm.at[src_v.at[i0]],
                                      rows_v.at[0], sem0)
                g1 = pltpu.async_copy(rows_hbm.at[src_v.at[i0 + 1]],
                                      rows_v.at[1], sem1)
                g0.wait()
                s0 = pltpu.async_copy(rows_v.at[0], acc.at[dst_v.at[i0]],
                                      sem2, add=True)
                g1.wait()
                s1 = pltpu.async_copy(rows_v.at[1],
                                      acc.at[dst_v.at[i0 + 1]], sem3,
                                      add=True)
                s0.wait()
                s1.wait()
                return carry

            lax.fori_loop(0, CH_W // 2, chunk_pair, 0)
            plsc.subcore_barrier()
            pltpu.sync_copy(
                acc.at[pl.ds(s * OROWS, OROWS)],
                out_hbm.at[c, pl.ds(s * OROWS, OROWS), pl.ds(k * HID, HID)])

            @pl.when(s == NS - 1)
            def _():
                pltpu.sync_copy(
                    acc.at[pl.ds(NS * OROWS, OREM)],
                    out_hbm.at[c, pl.ds(NS * OROWS, OREM),
                               pl.ds(k * HID, HID)])

            if k + 1 < nslabs:
                plsc.subcore_barrier()

    return _seg


_seg_sum = _make_seg_sum(1)


def _stage1_body(x_ref, s00, s01, s10, s11, s20, s21, w1_ref, b1_ref, w2_ref,
                 b2_ref, o_ref):
    agg = jnp.concatenate(
        [s00[...] + s01[...], s10[...] + s11[...], s20[...] + s21[...]],
        axis=1)
    hin = x_ref[...] + agg
    h = jnp.dot(hin, w1_ref[...], preferred_element_type=jnp.float32)
    h = jnp.maximum(h + b1_ref[...], 0.0)
    t = jnp.dot(h, w2_ref[...], preferred_element_type=jnp.float32)
    o_ref[...] = jnp.maximum(t + b2_ref[...], 0.0)


def _stage2_body(t_ref, a0_ref, a1_ref, w3_ref, b3_ref, w4_ref, b4_ref,
                 batch_ref, wc_ref, bc_ref, o_ref, acc_ref):
    i = pl.program_id(0)

    @pl.when(i == 0)
    def _():
        acc_ref[...] = jnp.zeros_like(acc_ref)

    hin = t_ref[...] + a0_ref[...] + a1_ref[...]
    h = jnp.dot(hin, w3_ref[...], preferred_element_type=jnp.float32)
    h = jnp.maximum(h + b3_ref[...], 0.0)
    t = jnp.dot(h, w4_ref[...], preferred_element_type=jnp.float32)
    t = jnp.maximum(t + b4_ref[...], 0.0)
    gids = lax.broadcasted_iota(jnp.int32, (NUM_GRAPHS, ROW_BLK), 0)
    oh = (jnp.broadcast_to(batch_ref[0], (NUM_GRAPHS, ROW_BLK)) == gids)
    pooled = lax.dot_general(oh.astype(jnp.float32), t,
                             (((1,), (0,)), ((), ())),
                             preferred_element_type=jnp.float32,
                             precision=lax.Precision.HIGHEST)
    acc_ref[...] += pooled

    @pl.when(i == N_BLKS - 1)
    def _():
        logits = jnp.dot(acc_ref[...], wc_ref[...],
                         preferred_element_type=jnp.float32) + bc_ref[...]
        o_ref[...] = jax.nn.sigmoid(logits)


def _stage1(x, slabs, w1, b1, w2, b2):
    row = pl.BlockSpec((ROW_BLK, HID), lambda i: (i, 0))
    return pl.pallas_call(
        _stage1_body,
        grid=(N_BLKS,),
        in_specs=[
            pl.BlockSpec((ROW_BLK, IN_DIM), lambda i: (i, 0)),
            row, row, row, row, row, row,
            pl.BlockSpec((IN_DIM, HID), lambda i: (0, 0)),
            pl.BlockSpec((HID,), lambda i: (0,)),
            pl.BlockSpec((HID, HID), lambda i: (0, 0)),
            pl.BlockSpec((HID,), lambda i: (0,)),
        ],
        out_specs=pl.BlockSpec((ROW_BLK, HID), lambda i: (i, 0)),
        out_shape=jax.ShapeDtypeStruct((N_NODES, HID), jnp.float32),
    )(x, slabs[0][0], slabs[0][1], slabs[1][0], slabs[1][1], slabs[2][0],
      slabs[2][1], w1, b1, w2, b2)


def _stage2(t1, a0, a1, w3, b3, w4, b4, batch3, wc, bc):
    row = pl.BlockSpec((ROW_BLK, HID), lambda i: (i, 0))
    full_v = pl.BlockSpec((HID,), lambda i: (0,))
    full_m = pl.BlockSpec((HID, HID), lambda i: (0, 0))
    return pl.pallas_call(
        _stage2_body,
        grid=(N_BLKS,),
        in_specs=[
            row, row, row, full_m, full_v, full_m, full_v,
            pl.BlockSpec((1, 1, ROW_BLK), lambda i: (i, 0, 0)),
            pl.BlockSpec((HID, NUM_SKILLS), lambda i: (0, 0)),
            pl.BlockSpec((NUM_SKILLS,), lambda i: (0,)),
        ],
        out_specs=pl.BlockSpec((NUM_GRAPHS, NUM_SKILLS), lambda i: (0, 0)),
        out_shape=jax.ShapeDtypeStruct((NUM_GRAPHS, NUM_SKILLS), jnp.float32),
        scratch_shapes=[pltpu.VMEM((NUM_GRAPHS, HID), jnp.float32)],
    )(t1, a0, a1, w3, b3, w4, b4, batch3, wc, bc)


def kernel(x, edge_index, batch, W1, b1, W2, b2, W3, b3, W4, b4, Wc, bc):
    x = x.astype(jnp.float32)
    src = edge_index[0].astype(jnp.int32)
    dst = edge_index[1].astype(jnp.int32)
    pad = EDGES_PAD - src.shape[0]
    src_p = jnp.concatenate([src, jnp.zeros((pad,), jnp.int32)])
    dst_p = jnp.concatenate([dst, jnp.full((pad,), DUMMY_ROW, jnp.int32)])
    dst_w = dst_p.reshape(NW, CH_W, CHUNK)
    src3 = 3 * src_p
    x3 = x.reshape(3 * N_NODES, HID)
    zeros = jnp.zeros((ZROWS, HID), jnp.float32)
    batch3 = batch.astype(jnp.int32).reshape(N_BLKS, 1, ROW_BLK)

    slabs = [_seg_sum(x3, (src3 + k).reshape(NW, 1, CH_W, CHUNK), dst_w,
                      zeros)
             for k in range(3)]
    t1 = _stage1(x, slabs, W1, b1, W2, b2)
    a2 = _seg_sum(t1, src_p.reshape(NW, 1, CH_W, CHUNK), dst_w, zeros)
    return _stage2(t1, a2[0], a2[1], W3, b3, W4, b4, batch3, Wc, bc)
